# Initial kernel scaffold; baseline (speedup 1.0000x reference)
import jax
import jax.numpy as jnp
from jax.experimental import pallas as pl


def kernel(x, edge_index, batch, W_embed, Ws, bs):
    raise NotImplementedError("write your pallas kernel here")


# speedup vs baseline: 10.7957x; 10.7957x over previous
import functools

import jax
import jax.numpy as jnp
from jax import lax
from jax.experimental import pallas as pl
from jax.experimental.pallas import tpu as pltpu
from jax.experimental.pallas import tpu_sc as plsc

N = 10000
E = 320000
G = 128
D = 128
N_LAYERS = 3

NW = 32
CHUNK = 128
CH_PER_TILE = 79
E_PAD = NW * CH_PER_TILE * CHUNK
N_ACC = N + 16

_R = 1000
_GRID = N // _R

def _sc_body(cur, src3, dst3, out, acc, src_v, dst_v, rows_v, sem):
    cid = lax.axis_index("c")
    sid = lax.axis_index("s")
    wid = sid * 2 + cid
    rows_per_tile = 624
    tail_base = 16 * rows_per_tile
    base = sid * rows_per_tile

    pltpu.sync_copy(cur.at[pl.ds(base, rows_per_tile)],
                    acc.at[pl.ds(base, rows_per_tile)])

    @pl.when(sid == 15)
    def _():
        pltpu.sync_copy(cur.at[pl.ds(tail_base, N - tail_base)],
                        acc.at[pl.ds(tail_base, N - tail_base)])
    pltpu.sync_copy(src3.at[wid], src_v)
    pltpu.sync_copy(dst3.at[wid], dst_v)
    plsc.subcore_barrier()

    def body(j, carry):
        pltpu.async_copy(cur.at[src_v.at[j]], rows_v, sem).wait()
        pltpu.sync_copy(rows_v, acc.at[dst_v.at[j]], add=True)
        return carry

    lax.fori_loop(0, CH_PER_TILE, body, 0)
    plsc.subcore_barrier()

    pltpu.sync_copy(acc.at[pl.ds(base, rows_per_tile)],
                    out.at[cid, pl.ds(base, rows_per_tile)])

    @pl.when(sid == 15)
    def _():
        pltpu.sync_copy(acc.at[pl.ds(tail_base, N - tail_base)],
                        out.at[cid, pl.ds(tail_base, N - tail_base)])


@functools.cache
def _sc_edge_agg_build():
    mesh = plsc.VectorSubcoreMesh(core_axis_name="c", subcore_axis_name="s")
    return pl.kernel(
        _sc_body,
        out_type=jax.ShapeDtypeStruct((2, N, D), jnp.float32),
        mesh=mesh,
        scratch_types=[
            pltpu.VMEM_SHARED((N_ACC, D), jnp.float32),
            pltpu.VMEM((CH_PER_TILE, CHUNK), jnp.int32),
            pltpu.VMEM((CH_PER_TILE, CHUNK), jnp.int32),
            pltpu.VMEM((CHUNK, D), jnp.float32),
            pltpu.SemaphoreType.DMA,
        ],
    )


def _sc_edge_agg(cur, src3, dst3):
    return _sc_edge_agg_build()(cur, src3, dst3)


def _pool_part(bt_ref, feat):
    b = bt_ref[0, 0, :]
    oh = (lax.broadcasted_iota(jnp.int32, (G, _R), 0) == b[None, :])
    return jnp.dot(oh.astype(jnp.float32), feat,
                   preferred_element_type=jnp.float32)


def _accum_pool(pool_ref, part):
    i = pl.program_id(0)

    @pl.when(i == 0)
    def _():
        pool_ref[...] = part

    @pl.when(i != 0)
    def _():
        pool_ref[...] = pool_ref[...] + part


def _embed_body(x_ref, w_ref, bt_ref, h_ref, pool_ref):
    h = jnp.dot(x_ref[...], w_ref[...], preferred_element_type=jnp.float32)
    h_ref[...] = h
    _accum_pool(pool_ref, _pool_part(bt_ref, h))


_embed_call = pl.pallas_call(
    _embed_body,
    grid=(_GRID,),
    in_specs=[
        pl.BlockSpec((_R, 32), lambda i: (i, 0)),
        pl.BlockSpec((32, D), lambda i: (0, 0)),
        pl.BlockSpec((1, 1, _R), lambda i: (i, 0, 0)),
    ],
    out_specs=[
        pl.BlockSpec((_R, D), lambda i: (i, 0)),
        pl.BlockSpec((G, D), lambda i: (0, 0)),
    ],
    out_shape=[
        jax.ShapeDtypeStruct((N, D), jnp.float32),
        jax.ShapeDtypeStruct((G, D), jnp.float32),
    ],
)


def _layer_body(a0_ref, a1_ref, w_ref, bias_ref, bt_ref, cur_ref, pool_ref):
    s = a0_ref[...] + a1_ref[...]
    cur = jnp.dot(s, w_ref[...], preferred_element_type=jnp.float32)
    cur = cur + bias_ref[...]
    cur_ref[...] = cur
    _accum_pool(pool_ref, _pool_part(bt_ref, cur))


_layer_call = pl.pallas_call(
    _layer_body,
    grid=(_GRID,),
    in_specs=[
        pl.BlockSpec((_R, D), lambda i: (i, 0)),
        pl.BlockSpec((_R, D), lambda i: (i, 0)),
        pl.BlockSpec((D, D), lambda i: (0, 0)),
        pl.BlockSpec((1, D), lambda i: (0, 0)),
        pl.BlockSpec((1, 1, _R), lambda i: (i, 0, 0)),
    ],
    out_specs=[
        pl.BlockSpec((_R, D), lambda i: (i, 0)),
        pl.BlockSpec((G, D), lambda i: (0, 0)),
    ],
    out_shape=[
        jax.ShapeDtypeStruct((N, D), jnp.float32),
        jax.ShapeDtypeStruct((G, D), jnp.float32),
    ],
)


def kernel(x, edge_index, batch, W_embed, Ws, bs):
    src = edge_index[0]
    dst = edge_index[1]
    pad = E_PAD - E
    pad_idx = jnp.arange(pad, dtype=jnp.int32)
    src3 = jnp.concatenate([src, pad_idx % N]).reshape(NW, CH_PER_TILE, CHUNK)
    dst3 = jnp.concatenate([dst, N + (pad_idx % 16)]).reshape(
        NW, CH_PER_TILE, CHUNK)
    batch3 = batch.reshape(_GRID, 1, _R)

    h, p0 = _embed_call(x, W_embed, batch3)
    pools = [p0]
    cur = h
    for i in range(N_LAYERS):
        agg = _sc_edge_agg(cur, src3, dst3)
        cur, p = _layer_call(agg[0], agg[1], Ws[i], bs[i].reshape(1, D),
                             batch3)
        pools.append(p)
    return jnp.concatenate(pools, axis=-1)


# speedup vs baseline: 16.0004x; 1.4821x over previous
import functools

import jax
import jax.numpy as jnp
from jax import lax
from jax.experimental import pallas as pl
from jax.experimental.pallas import tpu as pltpu
from jax.experimental.pallas import tpu_sc as plsc

N = 10000
E = 320000
G = 128
D = 128
N_LAYERS = 3

NW = 32
CHUNK = 128
CH_PER_TILE = 80
E_PAD = NW * CH_PER_TILE * CHUNK
N_ACC = N + 16
ROUNDS = CH_PER_TILE

_R = 1000
_GRID = N // _R

def _sc_body(cur, packed3, out, acc, idx_v, src_a, dst_a, src_b, dst_b,
             buf_a, buf_b, sem_a, sem_b):
    cid = lax.axis_index("c")
    sid = lax.axis_index("s")
    wid = sid * 2 + cid
    rows_per_tile = 624
    tail_base = 16 * rows_per_tile
    base = sid * rows_per_tile

    pltpu.sync_copy(cur.at[pl.ds(base, rows_per_tile)],
                    acc.at[pl.ds(base, rows_per_tile)])

    @pl.when(sid == 15)
    def _():
        pltpu.sync_copy(cur.at[pl.ds(tail_base, N - tail_base)],
                        acc.at[pl.ds(tail_base, N - tail_base)])
    pltpu.sync_copy(packed3.at[wid], idx_v)
    plsc.subcore_barrier()

    def unpack(r, src_buf, dst_buf):
        for i in range(CHUNK // 16):
            v = idx_v[r, pl.ds(i * 16, 16)]
            src_buf[pl.ds(i * 16, 16)] = v & 0x3FFF
            dst_buf[pl.ds(i * 16, 16)] = lax.shift_right_logical(v, 14)

    def fire(buf, sem, src_buf):
        pltpu.async_copy(cur.at[src_buf], buf, sem)

    def drain_scatter(buf, sem, dst_buf):
        pltpu.make_async_copy(cur.at[pl.ds(0, CHUNK)], buf, sem).wait()
        pltpu.sync_copy(buf, acc.at[dst_buf], add=True)

    unpack(0, src_a, dst_a)
    fire(buf_a, sem_a, src_a)
    unpack(1, src_b, dst_b)

    def body(o, carry):
        r = o * 2
        fire(buf_b, sem_b, src_b)
        drain_scatter(buf_a, sem_a, dst_a)

        @pl.when(o < ROUNDS // 2 - 1)
        def _():
            unpack(r + 2, src_a, dst_a)
            fire(buf_a, sem_a, src_a)

        drain_scatter(buf_b, sem_b, dst_b)

        @pl.when(o < ROUNDS // 2 - 1)
        def _():
            unpack(r + 3, src_b, dst_b)
        return carry

    lax.fori_loop(0, ROUNDS // 2, body, 0)
    plsc.subcore_barrier()

    pltpu.sync_copy(acc.at[pl.ds(base, rows_per_tile)],
                    out.at[cid, pl.ds(base, rows_per_tile)])

    @pl.when(sid == 15)
    def _():
        pltpu.sync_copy(acc.at[pl.ds(tail_base, N - tail_base)],
                        out.at[cid, pl.ds(tail_base, N - tail_base)])


@functools.cache
def _sc_edge_agg_build():
    mesh = plsc.VectorSubcoreMesh(core_axis_name="c", subcore_axis_name="s")
    return pl.kernel(
        _sc_body,
        out_type=jax.ShapeDtypeStruct((2, N, D), jnp.float32),
        mesh=mesh,
        scratch_types=[
            pltpu.VMEM_SHARED((N_ACC, D), jnp.float32),
            pltpu.VMEM((CH_PER_TILE, CHUNK), jnp.int32),
            pltpu.VMEM((CHUNK,), jnp.int32),
            pltpu.VMEM((CHUNK,), jnp.int32),
            pltpu.VMEM((CHUNK,), jnp.int32),
            pltpu.VMEM((CHUNK,), jnp.int32),
            pltpu.VMEM((CHUNK, D), jnp.float32),
            pltpu.VMEM((CHUNK, D), jnp.float32),
            pltpu.SemaphoreType.DMA,
            pltpu.SemaphoreType.DMA,
        ],
    )


def _sc_edge_agg(cur, packed3):
    return _sc_edge_agg_build()(cur, packed3)


def _pool_part(bt_ref, feat):
    b = bt_ref[0, 0, :]
    oh = (lax.broadcasted_iota(jnp.int32, (G, _R), 0) == b[None, :])
    return jnp.dot(oh.astype(jnp.float32), feat,
                   preferred_element_type=jnp.float32)


def _accum_pool(pool_ref, part):
    i = pl.program_id(0)

    @pl.when(i == 0)
    def _():
        pool_ref[...] = part

    @pl.when(i != 0)
    def _():
        pool_ref[...] = pool_ref[...] + part


def _embed_body(x_ref, w_ref, bt_ref, h_ref, pool_ref):
    h = jnp.dot(x_ref[...], w_ref[...], preferred_element_type=jnp.float32)
    h_ref[...] = h
    _accum_pool(pool_ref, _pool_part(bt_ref, h))


_embed_call = pl.pallas_call(
    _embed_body,
    grid=(_GRID,),
    in_specs=[
        pl.BlockSpec((_R, 32), lambda i: (i, 0)),
        pl.BlockSpec((32, D), lambda i: (0, 0)),
        pl.BlockSpec((1, 1, _R), lambda i: (i, 0, 0)),
    ],
    out_specs=[
        pl.BlockSpec((_R, D), lambda i: (i, 0)),
        pl.BlockSpec((G, D), lambda i: (0, 0)),
    ],
    out_shape=[
        jax.ShapeDtypeStruct((N, D), jnp.float32),
        jax.ShapeDtypeStruct((G, D), jnp.float32),
    ],
)


def _layer_body(a0_ref, a1_ref, w_ref, bias_ref, bt_ref, cur_ref, pool_ref):
    s = a0_ref[...] + a1_ref[...]
    cur = jnp.dot(s, w_ref[...], preferred_element_type=jnp.float32)
    cur = cur + bias_ref[...]
    cur_ref[...] = cur
    _accum_pool(pool_ref, _pool_part(bt_ref, cur))


_layer_call = pl.pallas_call(
    _layer_body,
    grid=(_GRID,),
    in_specs=[
        pl.BlockSpec((_R, D), lambda i: (i, 0)),
        pl.BlockSpec((_R, D), lambda i: (i, 0)),
        pl.BlockSpec((D, D), lambda i: (0, 0)),
        pl.BlockSpec((1, D), lambda i: (0, 0)),
        pl.BlockSpec((1, 1, _R), lambda i: (i, 0, 0)),
    ],
    out_specs=[
        pl.BlockSpec((_R, D), lambda i: (i, 0)),
        pl.BlockSpec((G, D), lambda i: (0, 0)),
    ],
    out_shape=[
        jax.ShapeDtypeStruct((N, D), jnp.float32),
        jax.ShapeDtypeStruct((G, D), jnp.float32),
    ],
)


def kernel(x, edge_index, batch, W_embed, Ws, bs):
    src = edge_index[0]
    dst = edge_index[1]
    pad = E_PAD - E
    pad_idx = jnp.arange(pad, dtype=jnp.int32)
    src_p = jnp.concatenate([src, pad_idx % N])
    dst_p = jnp.concatenate([dst, N + (pad_idx % 16)])
    packed3 = (src_p | (dst_p << 14)).reshape(NW, CH_PER_TILE, CHUNK)
    batch3 = batch.reshape(_GRID, 1, _R)

    h, p0 = _embed_call(x, W_embed, batch3)
    pools = [p0]
    cur = h
    for i in range(N_LAYERS):
        agg = _sc_edge_agg(cur, packed3)
        cur, p = _layer_call(agg[0], agg[1], Ws[i], bs[i].reshape(1, D),
                             batch3)
        pools.append(p)
    return jnp.concatenate(pools, axis=-1)
